# Initial kernel scaffold; baseline (speedup 1.0000x reference)
import jax
import jax.numpy as jnp
from jax.experimental import pallas as pl


def kernel(efeat, nfeat, edge_index, W_efeat, W_src, W_dst, b_in, W_out, b_out, ln_gamma, ln_beta):
    raise NotImplementedError("write your pallas kernel here")


# speedup vs baseline: 3.8958x; 3.8958x over previous
import functools

import jax
import jax.numpy as jnp
from jax import lax
from jax.experimental import pallas as pl
from jax.experimental.pallas import tpu as pltpu
from jax.experimental.pallas import tpu_sc as plsc


def _sc_gather(nfeat, src, dst):
    n, d = nfeat.shape
    e = src.shape[0]
    info = plsc.get_sparse_core_info()
    nw = info.num_cores * info.num_subcores
    per_w = e // nw
    ch = 400
    n_ch = per_w // ch
    assert per_w * nw == e and n_ch * ch == per_w

    mesh = plsc.VectorSubcoreMesh(core_axis_name="c", subcore_axis_name="s")

    @functools.partial(
        pl.kernel,
        out_type=(
            jax.ShapeDtypeStruct((e, d), jnp.float32),
            jax.ShapeDtypeStruct((e, d), jnp.float32),
        ),
        mesh=mesh,
        scratch_types=[
            pltpu.VMEM((per_w,), jnp.int32),
            pltpu.VMEM((per_w,), jnp.int32),
            pltpu.VMEM((ch, d), jnp.float32),
            pltpu.VMEM((ch, d), jnp.float32),
            pltpu.SemaphoreType.DMA,
            pltpu.SemaphoreType.DMA,
        ],
    )
    def gather_kernel(nfeat_hbm, src_hbm, dst_hbm, gsrc_hbm, gdst_hbm,
                      src_v, dst_v, rows_s, rows_d, sem_s, sem_d):
        wid = lax.axis_index("s") * info.num_cores + lax.axis_index("c")
        base = pl.multiple_of(wid * per_w, ch)
        pltpu.sync_copy(src_hbm.at[pl.ds(base, per_w)], src_v)
        pltpu.sync_copy(dst_hbm.at[pl.ds(base, per_w)], dst_v)

        def body(j, carry):
            off = pl.multiple_of(j * ch, ch)
            cp_s = pltpu.async_copy(
                nfeat_hbm.at[src_v.at[pl.ds(off, ch)]], rows_s, sem_s)
            cp_d = pltpu.async_copy(
                nfeat_hbm.at[dst_v.at[pl.ds(off, ch)]], rows_d, sem_d)
            cp_s.wait()
            pltpu.sync_copy(rows_s, gsrc_hbm.at[pl.ds(base + off, ch)])
            cp_d.wait()
            pltpu.sync_copy(rows_d, gdst_hbm.at[pl.ds(base + off, ch)])
            return carry

        lax.fori_loop(0, n_ch, body, 0)

    return gather_kernel(nfeat, src, dst)


def _tc_body(e_ref, s_ref, d_ref, we_ref, ws_ref, wd_ref, bin_ref,
             wo_ref, bo_ref, g_ref, b_ref, o_ref):
    x = (jnp.dot(e_ref[...], we_ref[...], preferred_element_type=jnp.float32)
         + jnp.dot(s_ref[...], ws_ref[...], preferred_element_type=jnp.float32)
         + jnp.dot(d_ref[...], wd_ref[...], preferred_element_type=jnp.float32)
         + bin_ref[...])
    h = x / (1.0 + jnp.exp(-x))
    out = jnp.dot(h, wo_ref[...], preferred_element_type=jnp.float32) + bo_ref[...]
    mean = jnp.mean(out, axis=-1, keepdims=True)
    var = jnp.mean((out - mean) ** 2, axis=-1, keepdims=True)
    o_ref[...] = (out - mean) * lax.rsqrt(var + 1e-5) * g_ref[...] + b_ref[...]


def _tc_fused(efeat, gsrc, gdst, we_t, ws_t, wd_t, b_in, wo_t, b_out,
              ln_gamma, ln_beta, block_e):
    e, d = efeat.shape
    h = we_t.shape[1]
    o = wo_t.shape[1]
    grid = (e // block_e,)
    row_spec = pl.BlockSpec((block_e, d), lambda i: (i, 0))
    full = lambda r, c: pl.BlockSpec((r, c), lambda i: (0, 0))
    return pl.pallas_call(
        _tc_body,
        grid=grid,
        in_specs=[
            row_spec, row_spec, row_spec,
            full(d, h), full(d, h), full(d, h), full(1, h),
            full(h, o), full(1, o), full(1, o), full(1, o),
        ],
        out_specs=pl.BlockSpec((block_e, o), lambda i: (i, 0)),
        out_shape=jax.ShapeDtypeStruct((e, o), jnp.float32),
        compiler_params=pltpu.CompilerParams(
            dimension_semantics=("arbitrary",)),
    )(efeat, gsrc, gdst, we_t, ws_t, wd_t, b_in, wo_t, b_out,
      ln_gamma, ln_beta)


def kernel(efeat, nfeat, edge_index, W_efeat, W_src, W_dst, b_in, W_out,
           b_out, ln_gamma, ln_beta):
    h = W_efeat.shape[0]
    o = W_out.shape[0]
    gsrc, gdst = _sc_gather(nfeat, edge_index[0], edge_index[1])
    return _tc_fused(
        efeat, gsrc, gdst,
        W_efeat.T, W_src.T, W_dst.T, b_in.reshape(1, h),
        W_out.T, b_out.reshape(1, o),
        ln_gamma.reshape(1, o), ln_beta.reshape(1, o),
        block_e=2560,
    )
